# Initial kernel scaffold; baseline (speedup 1.0000x reference)
import jax
import jax.numpy as jnp
from jax.experimental import pallas as pl


def kernel(x, edge_index, edge_weight, W, b):
    raise NotImplementedError("write your pallas kernel here")


# speedup vs baseline: 2.6576x; 2.6576x over previous
import functools

import jax
import jax.numpy as jnp
from jax import lax
from jax.experimental import pallas as pl
from jax.experimental.pallas import tpu as pltpu
from jax.experimental.pallas import tpu_sc as plsc

N = 10000
E = 320000
D = 128
H = D // 2
NC = 2
NS = 16
L = 16

EPT = E // NS
CH = 80
NCHUNK = EPT // CH
RPT = 624
RPT_LAST = N - (NS - 1) * RPT


def _linear_body(x_ref, wt_ref, b_ref, out_ref):
    h = jnp.dot(x_ref[...], wt_ref[...], preferred_element_type=jnp.float32)
    h = h + b_ref[...]
    out_ref[0] = h[:, :H]
    out_ref[1] = h[:, H:]


def _linear(x, wt, b2):
    blk = 1000
    grid = N // blk
    return pl.pallas_call(
        _linear_body,
        grid=(grid,),
        in_specs=[
            pl.BlockSpec((blk, D), lambda j: (j, 0)),
            pl.BlockSpec((D, D), lambda j: (0, 0)),
            pl.BlockSpec((1, D), lambda j: (0, 0)),
        ],
        out_specs=pl.BlockSpec((2, blk, H), lambda j: (0, j, 0)),
        out_shape=jax.ShapeDtypeStruct((2, N, H), jnp.float32),
    )(x, wt, b2)


def _lane_broadcast(v, e):
    idx = jnp.full((L, 1), e, dtype=jnp.int32)
    dnums = lax.GatherDimensionNumbers(
        offset_dims=(), collapsed_slice_dims=(0,), start_index_map=(0,))
    return lax.gather(v, idx, dnums, slice_sizes=(1,),
                      mode=lax.GatherScatterMode.PROMISE_IN_BOUNDS)


def _agg_body(hcat, row, col, ew, out,
              colbuf, rowbuf, wbuf, mbuf, rbuf, acc, sem):
    c = lax.axis_index("c")
    s = lax.axis_index("s")
    coff = (c * N).astype(jnp.int32)

    zeros = jnp.zeros((L,), jnp.float32)
    rstart = pl.multiple_of(s * RPT, 8)

    def zero_row(r, _):
        for k in range(H // L):
            rbuf[r, pl.ds(k * L, L)] = zeros
        return _
    lax.fori_loop(0, RPT_LAST, zero_row, None)

    @pl.when(s < NS - 1)
    def _():
        pltpu.sync_copy(rbuf.at[pl.ds(0, RPT)], acc.at[pl.ds(rstart, RPT)])

    @pl.when(s == NS - 1)
    def _():
        pltpu.sync_copy(rbuf.at[pl.ds(0, RPT_LAST)],
                        acc.at[pl.ds(rstart, RPT_LAST)])

    plsc.subcore_barrier()

    ebase = s * EPT

    def chunk_body(i, _):
        base = ebase + i * CH
        pltpu.sync_copy(col.at[pl.ds(base, CH)], colbuf)
        pltpu.sync_copy(row.at[pl.ds(base, CH)], rowbuf)
        pltpu.sync_copy(ew.at[pl.ds(base, CH)], wbuf)
        for g in range(CH // L):
            colbuf[pl.ds(g * L, L)] = colbuf[pl.ds(g * L, L)] + coff
        pltpu.async_copy(hcat.at[colbuf], mbuf, sem).wait()
        for g in range(CH // L):
            w16 = wbuf[pl.ds(g * L, L)]
            for e in range(L):
                wsplat = _lane_broadcast(w16, e)
                row_i = g * L + e
                for k in range(H // L):
                    mbuf[row_i, pl.ds(k * L, L)] = (
                        mbuf[row_i, pl.ds(k * L, L)] * wsplat)
        pltpu.sync_copy(mbuf, acc.at[rowbuf], add=True)
        return _

    lax.fori_loop(0, NCHUNK, chunk_body, None)
    plsc.subcore_barrier()

    def relu_stripe(nrows):
        pltpu.sync_copy(acc.at[pl.ds(rstart, nrows)],
                        rbuf.at[pl.ds(0, nrows)])

        def relu_row(r, _):
            for k in range(H // L):
                v = rbuf[r, pl.ds(k * L, L)]
                rbuf[r, pl.ds(k * L, L)] = jnp.where(v >= 0, v, v * 0.01)
            return _
        lax.fori_loop(0, nrows, relu_row, None)
        pltpu.sync_copy(rbuf.at[pl.ds(0, nrows)],
                        out.at[c, pl.ds(rstart, nrows)])

    @pl.when(s < NS - 1)
    def _():
        relu_stripe(RPT)

    @pl.when(s == NS - 1)
    def _():
        relu_stripe(RPT_LAST)


@functools.partial(
    pl.kernel,
    out_type=jax.ShapeDtypeStruct((NC, N, H), jnp.float32),
    mesh=plsc.VectorSubcoreMesh(core_axis_name="c", subcore_axis_name="s",
                                num_cores=NC, num_subcores=NS),
    compiler_params=pltpu.CompilerParams(use_tc_tiling_on_sc=False),
    scratch_types=[
        pltpu.VMEM((CH,), jnp.int32),
        pltpu.VMEM((CH,), jnp.int32),
        pltpu.VMEM((CH,), jnp.float32),
        pltpu.VMEM((CH, H), jnp.float32),
        pltpu.VMEM((RPT_LAST, H), jnp.float32),
        pltpu.VMEM_SHARED((N, H), jnp.float32),
        pltpu.SemaphoreType.DMA,
    ],
)
def _aggregate(hcat, row, col, ew, out, *scratch):
    _agg_body(hcat, row, col, ew, out, *scratch)


def kernel(x, edge_index, edge_weight, W, b):
    h2 = _linear(x, W.T, b.reshape(1, D))
    hcat = h2.reshape(2 * N, H)
    row = edge_index[0]
    col = edge_index[1]
    out3 = _aggregate(hcat, row, col, edge_weight)
    return jnp.concatenate([out3[0], out3[1]], axis=1)


# speedup vs baseline: 6.8830x; 2.5899x over previous
import functools

import jax
import jax.numpy as jnp
from jax import lax
from jax.experimental import pallas as pl
from jax.experimental.pallas import tpu as pltpu
from jax.experimental.pallas import tpu_sc as plsc

N = 10000
E = 320000
D = 128
H = D // 2
NC = 2
NS = 16
L = 16

EPT = E // NS
SB = 4000
NSC = EPT // SB
CH = 80
CPS = SB // CH
RPT = 624
RPT_LAST = N - (NS - 1) * RPT


def _linear_body(x_ref, wt_ref, b_ref, out_ref):
    h = jnp.dot(x_ref[...], wt_ref[...], preferred_element_type=jnp.float32)
    h = h + b_ref[...]
    out_ref[0] = h[:, :H]
    out_ref[1] = h[:, H:]


def _linear(x, wt, b2):
    blk = 1000
    grid = N // blk
    return pl.pallas_call(
        _linear_body,
        grid=(grid,),
        in_specs=[
            pl.BlockSpec((blk, D), lambda j: (j, 0)),
            pl.BlockSpec((D, D), lambda j: (0, 0)),
            pl.BlockSpec((1, D), lambda j: (0, 0)),
        ],
        out_specs=pl.BlockSpec((2, blk, H), lambda j: (0, j, 0)),
        out_shape=jax.ShapeDtypeStruct((2, N, H), jnp.float32),
    )(x, wt, b2)


def _lane_broadcast(v, e):
    idx = jnp.full((L, 1), e, dtype=jnp.int32)
    dnums = lax.GatherDimensionNumbers(
        offset_dims=(), collapsed_slice_dims=(0,), start_index_map=(0,))
    return lax.gather(v, idx, dnums, slice_sizes=(1,),
                      mode=lax.GatherScatterMode.PROMISE_IN_BOUNDS)


def _agg_body(hcat, row, col, ew, out,
              colsb, rowsb, wsb,
              colslot0, rowslot0, mbuf0,
              colslot1, rowslot1, mbuf1,
              rbuf, acc, gsem0, gsem1):
    c = lax.axis_index("c")
    s = lax.axis_index("s")
    coff = (c * N).astype(jnp.int32)
    slots = ((colslot0, rowslot0, mbuf0, gsem0),
             (colslot1, rowslot1, mbuf1, gsem1))

    zeros = jnp.zeros((L,), jnp.float32)
    rstart = pl.multiple_of(s * RPT, 8)

    def zero_row(r, _):
        for k in range(H // L):
            rbuf[r, pl.ds(k * L, L)] = zeros
        return _
    lax.fori_loop(0, RPT_LAST, zero_row, None)

    @pl.when(s < NS - 1)
    def _():
        pltpu.sync_copy(rbuf.at[pl.ds(0, RPT)], acc.at[pl.ds(rstart, RPT)])

    @pl.when(s == NS - 1)
    def _():
        pltpu.sync_copy(rbuf.at[pl.ds(0, RPT_LAST)],
                        acc.at[pl.ds(rstart, RPT_LAST)])

    plsc.subcore_barrier()

    ebase = s * EPT

    def load_slot(b, off):
        colslot, rowslot, _, _ = slots[b]
        for g in range(CH // L):
            colslot[pl.ds(g * L, L)] = colsb[pl.ds(off + g * L, L)]
            rowslot[pl.ds(g * L, L)] = rowsb[pl.ds(off + g * L, L)]

    def gather(b):
        colslot, _, mbuf, gsem = slots[b]
        return pltpu.make_async_copy(hcat.at[colslot], mbuf, gsem)

    def scale(b, off):
        _, _, mbuf, _ = slots[b]
        for g in range(CH // L):
            w16 = wsb[pl.ds(off + g * L, L)]
            for e in range(L):
                wsplat = _lane_broadcast(w16, e)
                r = g * L + e
                for k in range(H // L):
                    mbuf[r, pl.ds(k * L, L)] = mbuf[r, pl.ds(k * L, L)] * wsplat

    def scatter_add(b):
        _, rowslot, mbuf, _ = slots[b]
        pltpu.sync_copy(mbuf, acc.at[rowslot], add=True)

    for sc in range(NSC):
        sbase = ebase + sc * SB
        pltpu.sync_copy(col.at[pl.ds(sbase, SB)], colsb)
        pltpu.sync_copy(row.at[pl.ds(sbase, SB)], rowsb)
        pltpu.sync_copy(ew.at[pl.ds(sbase, SB)], wsb)

        def add_off(j, _):
            colsb[pl.ds(j * L, L)] = colsb[pl.ds(j * L, L)] + coff
            return _
        lax.fori_loop(0, SB // L, add_off, None)

        load_slot(0, 0)
        gather(0).start()
        load_slot(1, CH)
        gather(1).start()

        def pair_body(i, _):
            for b in range(2):
                off = (2 * i + b) * CH
                gather(b).wait()
                scale(b, off)
                scatter_add(b)
                load_slot(b, off + 2 * CH)
                gather(b).start()
            return _
        lax.fori_loop(0, (CPS - 2) // 2, pair_body, None)

        for b in range(2):
            off = (CPS - 2 + b) * CH
            gather(b).wait()
            scale(b, off)
            scatter_add(b)

    plsc.subcore_barrier()

    def relu_stripe(nrows):
        pltpu.sync_copy(acc.at[pl.ds(rstart, nrows)],
                        rbuf.at[pl.ds(0, nrows)])

        def relu_row(r, _):
            for k in range(H // L):
                v = rbuf[r, pl.ds(k * L, L)]
                rbuf[r, pl.ds(k * L, L)] = jnp.where(v >= 0, v, v * 0.01)
            return _
        lax.fori_loop(0, nrows, relu_row, None)
        pltpu.sync_copy(rbuf.at[pl.ds(0, nrows)],
                        out.at[c, pl.ds(rstart, nrows)])

    @pl.when(s < NS - 1)
    def _():
        relu_stripe(RPT)

    @pl.when(s == NS - 1)
    def _():
        relu_stripe(RPT_LAST)


@functools.partial(
    pl.kernel,
    out_type=jax.ShapeDtypeStruct((NC, N, H), jnp.float32),
    mesh=plsc.VectorSubcoreMesh(core_axis_name="c", subcore_axis_name="s",
                                num_cores=NC, num_subcores=NS),
    compiler_params=pltpu.CompilerParams(use_tc_tiling_on_sc=False),
    scratch_types=[
        pltpu.VMEM((SB,), jnp.int32),
        pltpu.VMEM((SB,), jnp.int32),
        pltpu.VMEM((SB,), jnp.float32),
        pltpu.VMEM((CH,), jnp.int32),
        pltpu.VMEM((CH,), jnp.int32),
        pltpu.VMEM((CH, H), jnp.float32),
        pltpu.VMEM((CH,), jnp.int32),
        pltpu.VMEM((CH,), jnp.int32),
        pltpu.VMEM((CH, H), jnp.float32),
        pltpu.VMEM((RPT_LAST, H), jnp.float32),
        pltpu.VMEM_SHARED((N, H), jnp.float32),
        pltpu.SemaphoreType.DMA,
        pltpu.SemaphoreType.DMA,
    ],
)
def _aggregate(hcat, row, col, ew, out, *scratch):
    _agg_body(hcat, row, col, ew, out, *scratch)


def kernel(x, edge_index, edge_weight, W, b):
    h2 = _linear(x, W.T, b.reshape(1, D))
    hcat = h2.reshape(2 * N, H)
    row = edge_index[0]
    col = edge_index[1]
    out3 = _aggregate(hcat, row, col, edge_weight)
    return jnp.concatenate([out3[0], out3[1]], axis=1)
